# baseline (speedup 1.0000x reference)
import numpy as np

import jax
import jax.numpy as jnp
from jax import lax
from jax.experimental import pallas as pl
from jax.experimental.pallas import tpu as pltpu
from jax.experimental.pallas import tpu_sc as plsc

BATCH = 16384
HIST = 50
D = 64
DW = D // 2
NUM_EMB = 1000000
B_TOT = BATCH * HIST
NW = 32
CHUNK = 128
N_BLOCKS = B_TOT // CHUNK
PER_W = N_BLOCKS // NW
SB = 4
N_SUPER = PER_W // SB
NBUF = 2
BT = BATCH // CHUNK

_MASK_ME = np.int32(0x0FFFE000)
_MASK_S = np.int32(-2147483648)
_SCALE = np.float32(2.0 ** 112)


def _emb_body(tbl_hbm, idx_hbm, out_hbm,
              idx_all, rows0, rows1, out0, out1,
              gsem0, gsem1, osem0, osem1):
    wid = lax.axis_index("s") * 2 + lax.axis_index("c")
    kbase = wid * PER_W
    lanes = lax.iota(jnp.int32, 16)

    rows = [rows0, rows1]
    outs = [out0, out1]
    gsems = [gsem0, gsem1]
    osems = [osem0, osem1]

    pltpu.sync_copy(idx_hbm.at[pl.ds(kbase, PER_W)], idx_all)

    def start_gathers(bb, sb):
        for j in range(SB):
            pltpu.async_copy(
                tbl_hbm.at[idx_all.at[sb * SB + j]],
                rows[bb].at[j], gsems[bb])

    def wait_gathers(bb):
        for j in range(SB):
            pltpu.make_async_copy(
                tbl_hbm.at[idx_all.at[j]], rows[bb].at[j], gsems[bb]).wait()

    def wait_outs(bb):
        for dh in range(D // 8):
            pltpu.make_async_copy(
                outs[bb].at[dh], out_hbm.at[0, dh, pl.ds(0, SB)],
                osems[bb]).wait()

    for bb in range(NBUF):
        start_gathers(bb, bb)

    def super_body(s2, carry):
        for bb in range(NBUF):
            s = s2 * NBUF + bb
            k0 = kbase + s * SB
            h = k0 // BT
            bt0 = k0 % BT
            wait_gathers(bb)

            @pl.when(s2 > 0)
            def _wait_out():
                wait_outs(bb)

            for j in range(SB):
                jj = jnp.full((16,), j, jnp.int32)

                def col_body(l0, carry2, _jj=jj, _j=j, _bb=bb):
                    ridx = l0 + lanes
                    for sw in range(DW):
                        w = plsc.load_gather(
                            rows[_bb],
                            [_jj, ridx, jnp.full((16,), sw, jnp.int32)])
                        f_lo = lax.bitcast_convert_type(
                            ((w << 13) & _MASK_ME) | ((w << 16) & _MASK_S),
                            jnp.float32) * _SCALE
                        f_hi = lax.bitcast_convert_type(
                            ((w >> 3) & _MASK_ME) | (w & _MASK_S),
                            jnp.float32) * _SCALE
                        d0 = 2 * sw
                        d1 = 2 * sw + 1
                        outs[_bb][d0 // 8, _j, d0 % 8, pl.ds(l0, 16)] = f_lo
                        outs[_bb][d1 // 8, _j, d1 % 8, pl.ds(l0, 16)] = f_hi
                    return carry2

                lax.fori_loop(0, CHUNK // 16, col_body, 0, unroll=False)

            for dh in range(D // 8):
                pltpu.async_copy(
                    outs[bb].at[dh], out_hbm.at[h, dh, pl.ds(bt0, SB)],
                    osems[bb])

            @pl.when(s2 < N_SUPER // NBUF - 1)
            def _next_gather():
                start_gathers(bb, s + NBUF)

        return carry

    lax.fori_loop(0, N_SUPER // NBUF, super_body, 0)

    for bb in range(NBUF):
        wait_outs(bb)


_emb = pl.kernel(
    _emb_body,
    out_type=jax.ShapeDtypeStruct((HIST, D // 8, BT, 8, CHUNK), jnp.float32),
    mesh=plsc.VectorSubcoreMesh(core_axis_name="c", subcore_axis_name="s"),
    compiler_params=pltpu.CompilerParams(
        needs_layout_passes=False, use_tc_tiling_on_sc=False),
    scratch_types=[
        pltpu.VMEM((PER_W, CHUNK), jnp.int32),
        pltpu.VMEM((SB, CHUNK, DW), jnp.int32),
        pltpu.VMEM((SB, CHUNK, DW), jnp.int32),
        pltpu.VMEM((D // 8, SB, 8, CHUNK), jnp.float32),
        pltpu.VMEM((D // 8, SB, 8, CHUNK), jnp.float32),
        pltpu.SemaphoreType.DMA,
        pltpu.SemaphoreType.DMA,
        pltpu.SemaphoreType.DMA,
        pltpu.SemaphoreType.DMA,
    ],
)


@jax.jit
def kernel(x, weight):
    idx = x.T.reshape(N_BLOCKS, CHUNK)
    w16 = jax.lax.bitcast_convert_type(weight.T, jnp.uint16)
    lo = w16[0::2, :].astype(jnp.uint32)
    hi = w16[1::2, :].astype(jnp.uint32)
    t32 = jax.lax.bitcast_convert_type(lo | (hi << 16), jnp.int32)
    tbl = t32.T
    out5 = _emb(tbl, idx)
    out = out5.transpose(2, 4, 0, 1, 3).reshape(BATCH, HIST, D)
    return out


# speedup vs baseline: 1.0507x; 1.0507x over previous
import numpy as np

import jax
import jax.numpy as jnp
from jax import lax
from jax.experimental import pallas as pl
from jax.experimental.pallas import tpu as pltpu
from jax.experimental.pallas import tpu_sc as plsc

BATCH = 16384
HIST = 50
D = 64
DW = D // 2
NUM_EMB = 1000000
B_TOT = BATCH * HIST
NW = 32
CHUNK = 128
N_BLOCKS = B_TOT // CHUNK
PER_W = N_BLOCKS // NW
SB = 4
N_SUPER = PER_W // SB
NBUF = 2
BT = BATCH // CHUNK
OBUF = SB * D * CHUNK
DH_RUN = SB * 8 * CHUNK

_MASK_ME = np.int32(0x0FFFE000)
_MASK_S = np.int32(-2147483648)
_SCALE = np.float32(2.0 ** 112)

def _emb_body(tbl_hbm, idx_hbm, out_hbm,
              idx_all, rows0, rows1, out0, out1,
              gsem0, gsem1, osem0, osem1):
    wid = lax.axis_index("s") * 2 + lax.axis_index("c")
    kbase = wid * PER_W

    rows = [rows0, rows1]
    outs = [out0, out1]
    gsems = [gsem0, gsem1]
    osems = [osem0, osem1]

    lanes2 = lax.iota(jnp.int32, 16) * 2
    scat = {}
    for s0 in range(2):
        for o in range(2):
            d = lanes2 + (32 * s0 + o)
            scat[(s0, o)] = (d >> 3) * DH_RUN + (d & 7) * CHUNK

    pltpu.sync_copy(idx_hbm.at[pl.ds(kbase, PER_W)], idx_all)

    def start_gathers(bb, sb):
        for j in range(SB):
            pltpu.async_copy(
                tbl_hbm.at[idx_all.at[sb * SB + j]],
                rows[bb].at[j], gsems[bb])

    def wait_gathers(bb):
        for j in range(SB):
            pltpu.make_async_copy(
                tbl_hbm.at[idx_all.at[j]], rows[bb].at[j], gsems[bb]).wait()

    def wait_outs(bb):
        for dh in range(D // 8):
            pltpu.make_async_copy(
                outs[bb].at[pl.ds(dh * DH_RUN, DH_RUN)],
                out_hbm.at[pl.ds(0, DH_RUN)], osems[bb]).wait()

    for bb in range(NBUF):
        start_gathers(bb, bb)

    def super_body(s2, carry):
        for bb in range(NBUF):
            s = s2 * NBUF + bb
            k0 = kbase + s * SB
            h = k0 // BT
            bt0 = k0 % BT
            wait_gathers(bb)

            @pl.when(s2 > 0)
            def _wait_out():
                wait_outs(bb)

            for j in range(SB):

                def row_body(r, carry2, _j=j, _bb=bb):
                    pos = (_j * 8 * CHUNK) + r
                    for s0 in range(2):
                        w = rows[_bb][_j, r, pl.ds(s0 * 16, 16)]
                        f_lo = lax.bitcast_convert_type(
                            ((w << 13) & _MASK_ME) | ((w << 16) & _MASK_S),
                            jnp.float32) * _SCALE
                        f_hi = lax.bitcast_convert_type(
                            ((w >> 3) & _MASK_ME) | (w & _MASK_S),
                            jnp.float32) * _SCALE
                        plsc.store_scatter(
                            outs[_bb], [scat[(s0, 0)] + pos], f_lo)
                        plsc.store_scatter(
                            outs[_bb], [scat[(s0, 1)] + pos], f_hi)
                    return carry2

                lax.fori_loop(0, CHUNK, row_body, 0, unroll=4)

            obase = h * (D * BATCH) + bt0 * (8 * CHUNK)
            for dh in range(D // 8):
                pltpu.async_copy(
                    outs[bb].at[pl.ds(dh * DH_RUN, DH_RUN)],
                    out_hbm.at[pl.ds(obase + dh * (8 * BATCH), DH_RUN)],
                    osems[bb])

            @pl.when(s2 < N_SUPER // NBUF - 1)
            def _next_gather():
                start_gathers(bb, s + NBUF)

        return carry

    lax.fori_loop(0, N_SUPER // NBUF, super_body, 0)

    for bb in range(NBUF):
        wait_outs(bb)


_emb = pl.kernel(
    _emb_body,
    out_type=jax.ShapeDtypeStruct((B_TOT * D,), jnp.float32),
    mesh=plsc.VectorSubcoreMesh(core_axis_name="c", subcore_axis_name="s"),
    compiler_params=pltpu.CompilerParams(
        needs_layout_passes=False, use_tc_tiling_on_sc=False),
    scratch_types=[
        pltpu.VMEM((PER_W, CHUNK), jnp.int32),
        pltpu.VMEM((SB, CHUNK, DW), jnp.int32),
        pltpu.VMEM((SB, CHUNK, DW), jnp.int32),
        pltpu.VMEM((OBUF,), jnp.float32),
        pltpu.VMEM((OBUF,), jnp.float32),
        pltpu.SemaphoreType.DMA,
        pltpu.SemaphoreType.DMA,
        pltpu.SemaphoreType.DMA,
        pltpu.SemaphoreType.DMA,
    ],
)


@jax.jit
def kernel(x, weight):
    idx = x.T.reshape(N_BLOCKS, CHUNK)
    w16 = jax.lax.bitcast_convert_type(weight, jnp.uint16)
    w2 = w16.reshape(NUM_EMB // 4, 4 * D)
    lo = w2[:, 0::2].astype(jnp.uint32)
    hi = w2[:, 1::2].astype(jnp.uint32)
    t32 = jax.lax.bitcast_convert_type(lo | (hi << 16), jnp.int32)
    tbl = t32.reshape(NUM_EMB, DW)
    flat = _emb(tbl, idx)
    out5 = flat.reshape(HIST, D // 8, BT, 8, CHUNK)
    out = out5.transpose(2, 4, 0, 1, 3).reshape(BATCH, HIST, D)
    return out


# speedup vs baseline: 7.4120x; 7.0545x over previous
import numpy as np

import jax
import jax.numpy as jnp
from jax import lax
from jax.experimental import pallas as pl
from jax.experimental.pallas import tpu as pltpu
from jax.experimental.pallas import tpu_sc as plsc

BATCH = 16384
HIST = 50
D = 64
DW = D // 2
NUM_EMB = 1000000
B_TOT = BATCH * HIST
NW = 32
CHUNK = 128
N_BLOCKS = B_TOT // CHUNK
PER_W = N_BLOCKS // NW
SB = 4
N_SUPER = PER_W // SB
NBUF = 2
BT = BATCH // CHUNK
OBUF = SB * D * CHUNK
DH_RUN = SB * 8 * CHUNK

_MASK_ME = np.int32(0x0FFFE000)
_MASK_S = np.int32(-2147483648)
_SCALE = np.float32(2.0 ** 112)

def _emb_body(tbl_hbm, idx_hbm, out_hbm,
              idx_all, rows0, rows1, out0, out1,
              gsem0, gsem1, osem0, osem1):
    wid = lax.axis_index("s") * 2 + lax.axis_index("c")
    kbase = wid * PER_W

    rows = [rows0, rows1]
    outs = [out0, out1]
    gsems = [gsem0, gsem1]
    osems = [osem0, osem1]

    lanes2 = lax.iota(jnp.int32, 16) * 2
    scat = {}
    for s0 in range(2):
        for o in range(2):
            d = lanes2 + (32 * s0 + o)
            scat[(s0, o)] = (d >> 3) * DH_RUN + (d & 7) * CHUNK

    pltpu.sync_copy(idx_hbm.at[pl.ds(kbase, PER_W)], idx_all)

    def start_gathers(bb, sb):
        for j in range(SB):
            pltpu.async_copy(
                tbl_hbm.at[idx_all.at[sb * SB + j]],
                rows[bb].at[j], gsems[bb])

    def wait_gathers(bb):
        for j in range(SB):
            pltpu.make_async_copy(
                tbl_hbm.at[idx_all.at[j]], rows[bb].at[j], gsems[bb]).wait()

    def wait_outs(bb):
        for dh in range(D // 8):
            pltpu.make_async_copy(
                outs[bb].at[pl.ds(dh * DH_RUN, DH_RUN)],
                out_hbm.at[pl.ds(0, DH_RUN)], osems[bb]).wait()

    for bb in range(NBUF):
        start_gathers(bb, bb)

    def super_body(s2, carry):
        for bb in range(NBUF):
            s = s2 * NBUF + bb
            k0 = kbase + s * SB
            h = k0 // BT
            bt0 = k0 % BT
            wait_gathers(bb)

            @pl.when(s2 > 0)
            def _wait_out():
                wait_outs(bb)

            for j in range(SB):

                def row_body(r, carry2, _j=j, _bb=bb):
                    pos = (_j * 8 * CHUNK) + r
                    for s0 in range(2):
                        w = rows[_bb][_j, r, pl.ds(s0 * 16, 16)]
                        f_lo = lax.bitcast_convert_type(
                            ((w << 13) & _MASK_ME) | ((w << 16) & _MASK_S),
                            jnp.float32) * _SCALE
                        f_hi = lax.bitcast_convert_type(
                            ((w >> 3) & _MASK_ME) | (w & _MASK_S),
                            jnp.float32) * _SCALE
                        plsc.store_scatter(
                            outs[_bb], [scat[(s0, 0)] + pos], f_lo)
                        plsc.store_scatter(
                            outs[_bb], [scat[(s0, 1)] + pos], f_hi)
                    return carry2

                lax.fori_loop(0, CHUNK, row_body, 0, unroll=4)

            obase = h * (D * BATCH) + bt0 * (8 * CHUNK)
            for dh in range(D // 8):
                pltpu.async_copy(
                    outs[bb].at[pl.ds(dh * DH_RUN, DH_RUN)],
                    out_hbm.at[pl.ds(obase + dh * (8 * BATCH), DH_RUN)],
                    osems[bb])

            @pl.when(s2 < N_SUPER // NBUF - 1)
            def _next_gather():
                start_gathers(bb, s + NBUF)

        return carry

    lax.fori_loop(0, N_SUPER // NBUF, super_body, 0)

    for bb in range(NBUF):
        wait_outs(bb)


_emb = pl.kernel(
    _emb_body,
    out_type=jax.ShapeDtypeStruct((B_TOT * D,), jnp.float32),
    mesh=plsc.VectorSubcoreMesh(core_axis_name="c", subcore_axis_name="s"),
    compiler_params=pltpu.CompilerParams(
        needs_layout_passes=False, use_tc_tiling_on_sc=False),
    scratch_types=[
        pltpu.VMEM((PER_W, CHUNK), jnp.int32),
        pltpu.VMEM((SB, CHUNK, DW), jnp.int32),
        pltpu.VMEM((SB, CHUNK, DW), jnp.int32),
        pltpu.VMEM((OBUF,), jnp.float32),
        pltpu.VMEM((OBUF,), jnp.float32),
        pltpu.SemaphoreType.DMA,
        pltpu.SemaphoreType.DMA,
        pltpu.SemaphoreType.DMA,
        pltpu.SemaphoreType.DMA,
    ],
)


_PACK_RO = 2048
_PACK_BC = _PACK_RO * 4
_PACK_GRID = -(-(NUM_EMB // 4) // _PACK_RO)


def _pack_body(x_ref, o_ref):
    w = pltpu.bitcast(x_ref[...], jnp.int32)
    for i in range(4):
        o_ref[:, i * 32:(i + 1) * 32] = w[:, i * _PACK_RO:(i + 1) * _PACK_RO].T


_pack = pl.pallas_call(
    _pack_body,
    grid=(_PACK_GRID,),
    in_specs=[pl.BlockSpec((D, _PACK_BC), lambda i: (0, i))],
    out_specs=pl.BlockSpec((_PACK_RO, 128), lambda i: (i, 0)),
    out_shape=jax.ShapeDtypeStruct((_PACK_GRID * _PACK_RO, 128), jnp.int32),
)


@jax.jit
def kernel(x, weight):
    r = x.T.reshape(N_BLOCKS, CHUNK)
    c = r & 8191
    idx = (r - c) + ((c & 2047) << 2) + (c >> 11)
    wt = jax.lax.bitcast_convert_type(weight, jnp.uint16).T
    tbl = _pack(wt).reshape(_PACK_GRID * _PACK_RO * 4, DW)
    flat = _emb(tbl, idx)
    out5 = flat.reshape(HIST, D // 8, BT, 8, CHUNK)
    out = out5.transpose(2, 4, 0, 1, 3).reshape(BATCH, HIST, D)
    return out
